# baseline (speedup 1.0000x reference)
import functools

import jax
import jax.numpy as jnp
from jax import lax
from jax.experimental import pallas as pl
from jax.experimental.pallas import tpu as pltpu

_NEG_POS = 3.0
_H = 512
_W = 512
_N = float(_H * _W)


def _sl1_of_mag(x):
    return jnp.where(x < 1.0, 0.5 * x * x, x - 0.5)


def _field_block(fields):
    ii = lax.broadcasted_iota(jnp.int32, (8, 128), 0)
    out = jnp.zeros((8, 128), jnp.float32)
    for r, f in enumerate(fields):
        out = out + jnp.where(ii == r, f, 0.0)
    return out


def _main_body(tril_ref, triu_ref, yt_ref, yp_ref, acc_ref):
    b = pl.program_id(0)
    c = pl.program_id(1)

    yt = yt_ref[0, 0]
    yp = yp_ref[0, 0]
    ad = jnp.abs(yt - yp)
    sl1 = _sl1_of_mag(ad)
    posb = yt >= 0.5
    posf = posb.astype(jnp.float32)
    z = jnp.logical_or(posb, ad == 0.0).astype(jnp.float32)
    psl1 = sl1 * posf

    ones_col = jnp.ones((_W, 1), jnp.float32)

    def _dot(a, bb):
        return jnp.dot(a, bb, preferred_element_type=jnp.float32)

    rz = _dot(z, ones_col)
    rp = _dot(psl1, ones_col)
    num_pos = jnp.sum(_dot(posf, ones_col))
    sl1_tot = jnp.sum(_dot(sl1, ones_col))
    pos_sl1 = jnp.sum(rp)
    S_nz = sl1_tot - pos_sl1
    nz = _N - jnp.sum(rz)
    k = jnp.minimum(_NEG_POS * num_pos, _N - 1.0)
    m = k - nz

    ro = jnp.floor(_dot(tril_ref[...], rz) + 0.5)

    full = ((ro + rz) <= m).astype(jnp.float32)
    rstar = jnp.sum(full)
    fullsum = jnp.sum(rp * full)

    ri = lax.broadcasted_iota(jnp.int32, (1, _H), 1)
    maskT = (ri == rstar.astype(jnp.int32)).astype(jnp.float32)
    zrow = _dot(maskT, z)
    prow = _dot(maskT, psl1)
    ro_r = _dot(maskT, ro)

    ew = jnp.floor(_dot(zrow, triu_ref[...]) + 0.5)
    partial = jnp.sum(prow * ((ro_r + ew) < m).astype(jnp.float32))

    negB = jnp.where(k > 0.0, S_nz + fullsum + partial, 0.0)
    needA = jnp.logical_and(k <= nz, k > 0.0).astype(jnp.float32)

    contrib = _field_block([num_pos, k, pos_sl1, negB, needA])

    @pl.when(jnp.logical_and(b == 0, c == 0))
    def _():
        acc_ref[...] = jnp.zeros_like(acc_ref)

    acc_ref[...] += contrib


def _fallback_body(yt_ref, yp_ref, acc_ref):
    b = pl.program_id(0)
    c = pl.program_id(1)

    yt = yt_ref[0, 0]
    yp = yp_ref[0, 0]
    ad = jnp.abs(yt - yp)
    sl1 = _sl1_of_mag(ad)
    posf = (yt >= 0.5).astype(jnp.float32)
    negf = 1.0 - posf
    loss = ad * negf
    floss = sl1 * negf

    num_pos = jnp.sum(posf)
    nz = jnp.sum((loss > 0.0).astype(jnp.float32))
    S_nz = jnp.sum(floss)
    k = jnp.minimum(_NEG_POS * num_pos, _N - 1.0)
    needA = jnp.logical_and(k <= nz, k > 0.0)

    bits = lax.bitcast_convert_type(loss, jnp.int32)

    def body(i, lo):
        cand = lo | (1 << (30 - i)).astype(jnp.int32)
        cnt = jnp.sum((bits >= cand).astype(jnp.float32))
        return jnp.where(cnt >= k, cand, lo)

    tbits = lax.fori_loop(0, 31, body, jnp.int32(0))
    t = lax.bitcast_convert_type(tbits, jnp.float32)
    gt = (bits > tbits).astype(jnp.float32)
    cnt_gt = jnp.sum(gt)
    sum_gt = jnp.sum(floss * gt)
    negA = sum_gt + (k - cnt_gt) * _sl1_of_mag(t)
    delta = jnp.where(needA, negA - S_nz, 0.0)

    contrib = _field_block([delta])

    @pl.when(jnp.logical_and(b == 0, c == 0))
    def _():
        acc_ref[...] = jnp.zeros_like(acc_ref)

    acc_ref[...] += contrib


def _plane_call(body, extra_specs, extra_args, y_true, y_pred):
    B, C, H, W = y_true.shape
    return pl.pallas_call(
        body,
        grid=(B, C),
        in_specs=extra_specs + [
            pl.BlockSpec((1, 1, H, W), lambda b, c: (b, c, 0, 0)),
            pl.BlockSpec((1, 1, H, W), lambda b, c: (b, c, 0, 0)),
        ],
        out_specs=pl.BlockSpec((8, 128), lambda b, c: (0, 0)),
        out_shape=jax.ShapeDtypeStruct((8, 128), jnp.float32),
        compiler_params=pltpu.CompilerParams(
            dimension_semantics=("arbitrary", "arbitrary")),
    )(*extra_args, y_true, y_pred)


@jax.jit
def kernel(y_true, y_pred):
    ii = lax.broadcasted_iota(jnp.int32, (_H, _H), 0)
    jj = lax.broadcasted_iota(jnp.int32, (_H, _H), 1)
    tril = (jj < ii).astype(jnp.float32)
    tri_spec = pl.BlockSpec((_H, _H), lambda b, c: (0, 0))
    acc = _plane_call(_main_body, [tri_spec, tri_spec], [tril, tril.T],
                      y_true, y_pred)
    pos_cnt = jnp.maximum(acc[0, 0], 1.0)
    neg_cnt = jnp.maximum(acc[1, 0], 1.0)
    delta = lax.cond(
        acc[4, 0] > 0.5,
        lambda: _plane_call(_fallback_body, [], [], y_true, y_pred)[0, 0],
        lambda: jnp.float32(0.0),
    )
    return _NEG_POS * acc[2, 0] / pos_cnt + (acc[3, 0] + delta) / neg_cnt


# speedup vs baseline: 1.1938x; 1.1938x over previous
import functools

import jax
import jax.numpy as jnp
from jax import lax
from jax.experimental import pallas as pl
from jax.experimental.pallas import tpu as pltpu

_NEG_POS = 3.0
_H = 512
_W = 512
_N = float(_H * _W)


def _sl1_of_mag(x):
    return jnp.where(x < 1.0, 0.5 * x * x, x - 0.5)


def _field_block(fields):
    ii = lax.broadcasted_iota(jnp.int32, (8, 128), 0)
    out = jnp.zeros((8, 128), jnp.float32)
    for r, f in enumerate(fields):
        out = out + jnp.where(ii == r, f, 0.0)
    return out


def _main_body(tril_ref, triu_ref, yt_ref, yp_ref, acc_ref):
    b = pl.program_id(0)
    c = pl.program_id(1)

    yt = yt_ref[0, 0]
    yp = yp_ref[0, 0]
    ad = jnp.abs(yt - yp)
    sl1 = _sl1_of_mag(ad)
    posb = yt >= 0.5
    posf = posb.astype(jnp.float32)
    z = jnp.logical_or(posb, ad == 0.0).astype(jnp.float32)
    psl1 = sl1 * posf

    def _dot(a, bb):
        return jnp.dot(a, bb, preferred_element_type=jnp.float32)

    rz = jnp.sum(z, axis=1, keepdims=True)
    rp = jnp.sum(psl1, axis=1, keepdims=True)
    num_pos = jnp.sum(posf)
    sl1_tot = jnp.sum(sl1)
    pos_sl1 = jnp.sum(rp)
    S_nz = sl1_tot - pos_sl1
    nz = _N - jnp.sum(rz)
    k = jnp.minimum(_NEG_POS * num_pos, _N - 1.0)
    m = k - nz

    ro = jnp.floor(_dot(tril_ref[...], rz) + 0.5)

    full = ((ro + rz) <= m).astype(jnp.float32)
    rstar = jnp.sum(full)
    fullsum = jnp.sum(rp * full)

    ri = lax.broadcasted_iota(jnp.int32, (_H, 1), 0)
    rowmask = (ri == rstar.astype(jnp.int32)).astype(jnp.float32)
    zrow = jnp.sum(z * rowmask, axis=0, keepdims=True)
    prow = jnp.sum(psl1 * rowmask, axis=0, keepdims=True)
    ro_r = jnp.sum(ro * rowmask)

    ew = jnp.floor(_dot(zrow, triu_ref[...]) + 0.5)
    partial = jnp.sum(prow * ((ro_r + ew) < m).astype(jnp.float32))

    negB = jnp.where(k > 0.0, S_nz + fullsum + partial, 0.0)
    needA = jnp.logical_and(k <= nz, k > 0.0).astype(jnp.float32)

    contrib = _field_block([num_pos, k, pos_sl1, negB, needA])

    @pl.when(jnp.logical_and(b == 0, c == 0))
    def _():
        acc_ref[...] = jnp.zeros_like(acc_ref)

    acc_ref[...] += contrib


def _fallback_body(yt_ref, yp_ref, acc_ref):
    b = pl.program_id(0)
    c = pl.program_id(1)

    yt = yt_ref[0, 0]
    yp = yp_ref[0, 0]
    ad = jnp.abs(yt - yp)
    sl1 = _sl1_of_mag(ad)
    posf = (yt >= 0.5).astype(jnp.float32)
    negf = 1.0 - posf
    loss = ad * negf
    floss = sl1 * negf

    num_pos = jnp.sum(posf)
    nz = jnp.sum((loss > 0.0).astype(jnp.float32))
    S_nz = jnp.sum(floss)
    k = jnp.minimum(_NEG_POS * num_pos, _N - 1.0)
    needA = jnp.logical_and(k <= nz, k > 0.0)

    bits = lax.bitcast_convert_type(loss, jnp.int32)

    def body(i, lo):
        cand = lo | (1 << (30 - i)).astype(jnp.int32)
        cnt = jnp.sum((bits >= cand).astype(jnp.float32))
        return jnp.where(cnt >= k, cand, lo)

    tbits = lax.fori_loop(0, 31, body, jnp.int32(0))
    t = lax.bitcast_convert_type(tbits, jnp.float32)
    gt = (bits > tbits).astype(jnp.float32)
    cnt_gt = jnp.sum(gt)
    sum_gt = jnp.sum(floss * gt)
    negA = sum_gt + (k - cnt_gt) * _sl1_of_mag(t)
    delta = jnp.where(needA, negA - S_nz, 0.0)

    contrib = _field_block([delta])

    @pl.when(jnp.logical_and(b == 0, c == 0))
    def _():
        acc_ref[...] = jnp.zeros_like(acc_ref)

    acc_ref[...] += contrib


def _plane_call(body, extra_specs, extra_args, y_true, y_pred):
    B, C, H, W = y_true.shape
    return pl.pallas_call(
        body,
        grid=(B, C),
        in_specs=extra_specs + [
            pl.BlockSpec((1, 1, H, W), lambda b, c: (b, c, 0, 0)),
            pl.BlockSpec((1, 1, H, W), lambda b, c: (b, c, 0, 0)),
        ],
        out_specs=pl.BlockSpec((8, 128), lambda b, c: (0, 0)),
        out_shape=jax.ShapeDtypeStruct((8, 128), jnp.float32),
        compiler_params=pltpu.CompilerParams(
            dimension_semantics=("arbitrary", "arbitrary")),
    )(*extra_args, y_true, y_pred)


@jax.jit
def kernel(y_true, y_pred):
    ii = lax.broadcasted_iota(jnp.int32, (_H, _H), 0)
    jj = lax.broadcasted_iota(jnp.int32, (_H, _H), 1)
    tril = (jj < ii).astype(jnp.float32)
    tri_spec = pl.BlockSpec((_H, _H), lambda b, c: (0, 0))
    acc = _plane_call(_main_body, [tri_spec, tri_spec], [tril, tril.T],
                      y_true, y_pred)
    pos_cnt = jnp.maximum(acc[0, 0], 1.0)
    neg_cnt = jnp.maximum(acc[1, 0], 1.0)
    delta = lax.cond(
        acc[4, 0] > 0.5,
        lambda: _plane_call(_fallback_body, [], [], y_true, y_pred)[0, 0],
        lambda: jnp.float32(0.0),
    )
    return _NEG_POS * acc[2, 0] / pos_cnt + (acc[3, 0] + delta) / neg_cnt


# speedup vs baseline: 1.2704x; 1.0641x over previous
import functools

import jax
import jax.numpy as jnp
from jax import lax
from jax.experimental import pallas as pl
from jax.experimental.pallas import tpu as pltpu

_NEG_POS = 3.0
_H = 512
_W = 512
_N = float(_H * _W)


def _sl1_of_mag(x):
    return jnp.where(x < 1.0, 0.5 * x * x, x - 0.5)


def _field_block(fields):
    ii = lax.broadcasted_iota(jnp.int32, (8, 128), 0)
    out = jnp.zeros((8, 128), jnp.float32)
    for r, f in enumerate(fields):
        out = out + jnp.where(ii == r, f, 0.0)
    return out


def _main_body(tril_ref, triu_ref, yt_ref, yp_ref, acc_ref):
    b = pl.program_id(0)
    c = pl.program_id(1)

    yt = yt_ref[0, 0]
    yp = yp_ref[0, 0]
    ad = jnp.abs(yt - yp)
    sl1 = _sl1_of_mag(ad)
    posb = yt >= 0.5
    posf = posb.astype(jnp.float32)
    z = jnp.logical_or(posb, ad == 0.0).astype(jnp.float32)
    psl1 = sl1 * posf

    def _dot(a, bb):
        return jnp.dot(a, bb, preferred_element_type=jnp.float32)

    rz = jnp.sum(z, axis=1, keepdims=True)
    rp = jnp.sum(psl1, axis=1, keepdims=True)
    num_pos = jnp.sum(posf)
    sl1_tot = jnp.sum(sl1)
    pos_sl1 = jnp.sum(rp)
    S_nz = sl1_tot - pos_sl1
    nz = _N - jnp.sum(rz)
    k = jnp.minimum(_NEG_POS * num_pos, _N - 1.0)
    m = k - nz

    ro = jnp.floor(_dot(tril_ref[...], rz) + 0.5)

    full = ((ro + rz) <= m).astype(jnp.float32)
    rstar = jnp.sum(full)
    fullsum = jnp.sum(rp * full)

    ri = lax.broadcasted_iota(jnp.int32, (_H, 1), 0)
    rstar_i = jnp.clip(rstar.astype(jnp.int32), 0, _H - 1)
    ytr = yt_ref[0, 0, pl.ds(rstar_i, 1), :]
    ypr = yp_ref[0, 0, pl.ds(rstar_i, 1), :]
    adr = jnp.abs(ytr - ypr)
    posbr = ytr >= 0.5
    zrow = jnp.logical_or(posbr, adr == 0.0).astype(jnp.float32)
    prow = jnp.where(posbr, _sl1_of_mag(adr), 0.0)
    rowmask = (ri == rstar_i).astype(jnp.float32)
    ro_r = jnp.sum(ro * rowmask)

    ew = jnp.floor(_dot(zrow, triu_ref[...]) + 0.5)
    partial = jnp.sum(prow * ((ro_r + ew) < m).astype(jnp.float32))

    negB = jnp.where(k > 0.0, S_nz + fullsum + partial, 0.0)
    needA = jnp.logical_and(k <= nz, k > 0.0).astype(jnp.float32)

    contrib = _field_block([num_pos, k, pos_sl1, negB, needA])

    @pl.when(jnp.logical_and(b == 0, c == 0))
    def _():
        acc_ref[...] = jnp.zeros_like(acc_ref)

    acc_ref[...] += contrib


def _fallback_body(yt_ref, yp_ref, acc_ref):
    b = pl.program_id(0)
    c = pl.program_id(1)

    yt = yt_ref[0, 0]
    yp = yp_ref[0, 0]
    ad = jnp.abs(yt - yp)
    sl1 = _sl1_of_mag(ad)
    posf = (yt >= 0.5).astype(jnp.float32)
    negf = 1.0 - posf
    loss = ad * negf
    floss = sl1 * negf

    num_pos = jnp.sum(posf)
    nz = jnp.sum((loss > 0.0).astype(jnp.float32))
    S_nz = jnp.sum(floss)
    k = jnp.minimum(_NEG_POS * num_pos, _N - 1.0)
    needA = jnp.logical_and(k <= nz, k > 0.0)

    bits = lax.bitcast_convert_type(loss, jnp.int32)

    def body(i, lo):
        cand = lo | (1 << (30 - i)).astype(jnp.int32)
        cnt = jnp.sum((bits >= cand).astype(jnp.float32))
        return jnp.where(cnt >= k, cand, lo)

    tbits = lax.fori_loop(0, 31, body, jnp.int32(0))
    t = lax.bitcast_convert_type(tbits, jnp.float32)
    gt = (bits > tbits).astype(jnp.float32)
    cnt_gt = jnp.sum(gt)
    sum_gt = jnp.sum(floss * gt)
    negA = sum_gt + (k - cnt_gt) * _sl1_of_mag(t)
    delta = jnp.where(needA, negA - S_nz, 0.0)

    contrib = _field_block([delta])

    @pl.when(jnp.logical_and(b == 0, c == 0))
    def _():
        acc_ref[...] = jnp.zeros_like(acc_ref)

    acc_ref[...] += contrib


def _plane_call(body, extra_specs, extra_args, y_true, y_pred):
    B, C, H, W = y_true.shape
    return pl.pallas_call(
        body,
        grid=(B, C),
        in_specs=extra_specs + [
            pl.BlockSpec((1, 1, H, W), lambda b, c: (b, c, 0, 0)),
            pl.BlockSpec((1, 1, H, W), lambda b, c: (b, c, 0, 0)),
        ],
        out_specs=pl.BlockSpec((8, 128), lambda b, c: (0, 0)),
        out_shape=jax.ShapeDtypeStruct((8, 128), jnp.float32),
        compiler_params=pltpu.CompilerParams(
            dimension_semantics=("arbitrary", "arbitrary")),
    )(*extra_args, y_true, y_pred)


@jax.jit
def kernel(y_true, y_pred):
    ii = lax.broadcasted_iota(jnp.int32, (_H, _H), 0)
    jj = lax.broadcasted_iota(jnp.int32, (_H, _H), 1)
    tril = (jj < ii).astype(jnp.float32)
    tri_spec = pl.BlockSpec((_H, _H), lambda b, c: (0, 0))
    acc = _plane_call(_main_body, [tri_spec, tri_spec], [tril, tril.T],
                      y_true, y_pred)
    pos_cnt = jnp.maximum(acc[0, 0], 1.0)
    neg_cnt = jnp.maximum(acc[1, 0], 1.0)
    delta = lax.cond(
        acc[4, 0] > 0.5,
        lambda: _plane_call(_fallback_body, [], [], y_true, y_pred)[0, 0],
        lambda: jnp.float32(0.0),
    )
    return _NEG_POS * acc[2, 0] / pos_cnt + (acc[3, 0] + delta) / neg_cnt
